# Initial kernel scaffold; baseline (speedup 1.0000x reference)
import jax
import jax.numpy as jnp
from jax.experimental import pallas as pl


def kernel(x, edge_index, edge_attr, eps, We, be, W1, b1, W2, b2, gamma, beta):
    raise NotImplementedError("write your pallas kernel here")


# speedup vs baseline: 3.2102x; 3.2102x over previous
import functools

import jax
import jax.numpy as jnp
from jax import lax
from jax.experimental import pallas as pl
from jax.experimental.pallas import tpu as pltpu
from jax.experimental.pallas import tpu_sc as plsc

NC = 2
NS = 16
LANES = 16
CH = 80
ZR = 32


def _sc_aggregate(x, ei, ej, attr, n_nodes, d):
    e = ei.shape[0]
    ed = attr.shape[1]
    nw = NC * NS
    epw = e // nw
    nch = epw // CH
    npad = -(-n_nodes // (NS * ZR)) * (NS * ZR)
    rpt = npad // NS

    mesh = plsc.VectorSubcoreMesh(core_axis_name="c", subcore_axis_name="s")

    @functools.partial(
        pl.kernel,
        out_type=(
            jax.ShapeDtypeStruct((NC, npad, d), jnp.float32),
            jax.ShapeDtypeStruct((NC, npad, ed), jnp.float32),
            jax.ShapeDtypeStruct((NC, npad, LANES), jnp.float32),
        ),
        mesh=mesh,
        compiler_params=pltpu.CompilerParams(use_tc_tiling_on_sc=False),
        scratch_types=[
            pltpu.VMEM_SHARED((npad, d), jnp.float32),
            pltpu.VMEM_SHARED((npad, ed), jnp.float32),
            pltpu.VMEM_SHARED((npad, LANES), jnp.float32),
            pltpu.VMEM((CH,), jnp.int32),
            pltpu.VMEM((CH,), jnp.int32),
            pltpu.VMEM((CH, d), jnp.float32),
            pltpu.VMEM((CH, ed), jnp.float32),
            pltpu.VMEM((CH, LANES), jnp.float32),
            pltpu.VMEM((ZR, d), jnp.float32),
            pltpu.VMEM((ZR, ed), jnp.float32),
            pltpu.SemaphoreType.DMA,
        ],
    )
    def sc_fn(x_hbm, i_hbm, j_hbm, attr_hbm, ax_out, s_out, deg_out,
              ax_acc, s_acc, deg_acc, ibuf, jbuf, rows, attrbuf, onesbuf,
              zbuf, zbuf_s, sem):
        cid = lax.axis_index("c")
        sid = lax.axis_index("s")
        wid = cid * NS + sid

        zero16 = jnp.zeros((LANES,), jnp.float32)
        ones16 = jnp.ones((LANES,), jnp.float32)

        def zstore(t, carry):
            r = t // (d // LANES)
            c = (t % (d // LANES)) * LANES
            zbuf[r, pl.ds(c, LANES)] = zero16
            return carry

        lax.fori_loop(0, ZR * (d // LANES), zstore, 0)

        def zstore_s(r, carry):
            zbuf_s[r, pl.ds(0, LANES)] = zero16
            return carry

        lax.fori_loop(0, ZR, zstore_s, 0)

        def ostore(r, carry):
            onesbuf[r, pl.ds(0, LANES)] = ones16
            return carry

        lax.fori_loop(0, CH, ostore, 0)

        r0 = sid * rpt
        for z in range(rpt // ZR):
            pltpu.sync_copy(zbuf, ax_acc.at[pl.ds(r0 + z * ZR, ZR), :])
            pltpu.sync_copy(zbuf_s, s_acc.at[pl.ds(r0 + z * ZR, ZR), :])
            pltpu.sync_copy(zbuf_s, deg_acc.at[pl.ds(r0 + z * ZR, ZR), :])

        plsc.subcore_barrier()

        ebase = wid * epw

        def chunk(k, carry):
            off = ebase + k * CH
            pltpu.sync_copy(i_hbm.at[pl.ds(off, CH)], ibuf)
            pltpu.sync_copy(j_hbm.at[pl.ds(off, CH)], jbuf)
            pltpu.async_copy(x_hbm.at[jbuf], rows, sem).wait()
            pltpu.sync_copy(attr_hbm.at[pl.ds(off, CH), :], attrbuf)
            pltpu.sync_copy(rows, ax_acc.at[ibuf], add=True)
            pltpu.sync_copy(attrbuf, s_acc.at[ibuf], add=True)
            pltpu.sync_copy(onesbuf, deg_acc.at[ibuf], add=True)
            return carry

        lax.fori_loop(0, nch, chunk, 0)

        plsc.subcore_barrier()

        pltpu.sync_copy(ax_acc.at[pl.ds(r0, rpt), :],
                        ax_out.at[cid, pl.ds(r0, rpt), :])
        pltpu.sync_copy(s_acc.at[pl.ds(r0, rpt), :],
                        s_out.at[cid, pl.ds(r0, rpt), :])
        pltpu.sync_copy(deg_acc.at[pl.ds(r0, rpt), :],
                        deg_out.at[cid, pl.ds(r0, rpt), :])

    return sc_fn(x, ei, ej, attr)


def _tc_dense(eps2, x, axp, sp, degp, we_t, be_r, w1_t, b1r, w2_t, b2r,
              g2, bt2):
    n, d = x.shape
    ed = we_t.shape[0]
    bn = 1000

    def body(eps_ref, x_ref, ax_ref, s_ref, deg_ref, we_ref, be_ref,
             w1_ref, b1_ref, w2_ref, b2_ref, g_ref, bt_ref, o_ref):
        ax = ax_ref[0] + ax_ref[1]
        s = s_ref[0] + s_ref[1]
        deg = (deg_ref[0] + deg_ref[1])[:, 0:1]
        agg = ax + jnp.dot(s, we_ref[...], preferred_element_type=jnp.float32)
        agg = agg + deg * be_ref[...]
        h = x_ref[...] * (1.0 + eps_ref[...]) + agg
        h = jnp.dot(h, w1_ref[...], preferred_element_type=jnp.float32)
        h = jnp.maximum(h + b1_ref[...], 0.0)
        h = jnp.dot(h, w2_ref[...], preferred_element_type=jnp.float32)
        h = h + b2_ref[...]
        mu = jnp.mean(h, axis=-1, keepdims=True)
        hc = h - mu
        var = jnp.mean(hc * hc, axis=-1, keepdims=True)
        o_ref[...] = hc * lax.rsqrt(var + 1e-5) * g_ref[...] + bt_ref[...]

    return pl.pallas_call(
        body,
        grid=(n // bn,),
        in_specs=[
            pl.BlockSpec((1, 1), lambda i: (0, 0)),
            pl.BlockSpec((bn, d), lambda i: (i, 0)),
            pl.BlockSpec((NC, bn, d), lambda i: (0, i, 0)),
            pl.BlockSpec((NC, bn, ed), lambda i: (0, i, 0)),
            pl.BlockSpec((NC, bn, LANES), lambda i: (0, i, 0)),
            pl.BlockSpec((ed, d), lambda i: (0, 0)),
            pl.BlockSpec((1, d), lambda i: (0, 0)),
            pl.BlockSpec((d, d), lambda i: (0, 0)),
            pl.BlockSpec((1, d), lambda i: (0, 0)),
            pl.BlockSpec((d, d), lambda i: (0, 0)),
            pl.BlockSpec((1, d), lambda i: (0, 0)),
            pl.BlockSpec((1, d), lambda i: (0, 0)),
            pl.BlockSpec((1, d), lambda i: (0, 0)),
        ],
        out_specs=pl.BlockSpec((bn, d), lambda i: (i, 0)),
        out_shape=jax.ShapeDtypeStruct((n, d), jnp.float32),
    )(eps2, x, axp, sp, degp, we_t, be_r, w1_t, b1r, w2_t, b2r, g2, bt2)


def kernel(x, edge_index, edge_attr, eps, We, be, W1, b1, W2, b2, gamma, beta):
    n, d = x.shape

    ei = edge_index[0].astype(jnp.int32)
    ej = edge_index[1].astype(jnp.int32)
    attr = edge_attr.astype(jnp.float32)

    axp, sp, degp = _sc_aggregate(x, ei, ej, attr, n, d)

    eps2 = jnp.reshape(eps.astype(jnp.float32), (1, 1))
    return _tc_dense(eps2, x, axp, sp, degp, We.T, be[None, :],
                     W1.T, b1[None, :], W2.T, b2[None, :],
                     gamma[None, :], beta[None, :])


# speedup vs baseline: 5.5615x; 1.7325x over previous
import functools

import jax
import jax.numpy as jnp
from jax import lax
from jax.experimental import pallas as pl
from jax.experimental.pallas import tpu as pltpu
from jax.experimental.pallas import tpu_sc as plsc

NC = 2
NS = 16
LANES = 16
CH = 80


def _sc_aggregate(x, ei, ej, attr, n_nodes, d):
    e = ei.shape[0]
    ed = attr.shape[1]
    nw = NC * NS
    epw = e // nw
    nch = epw // CH
    npad = -(-n_nodes // (NS * CH)) * (NS * CH)
    rpt = npad // NS

    mesh = plsc.VectorSubcoreMesh(core_axis_name="c", subcore_axis_name="s")

    @functools.partial(
        pl.kernel,
        out_type=(
            jax.ShapeDtypeStruct((NC, npad, d), jnp.float32),
            jax.ShapeDtypeStruct((NC, npad, ed), jnp.float32),
            jax.ShapeDtypeStruct((NC, npad, LANES), jnp.float32),
        ),
        mesh=mesh,
        compiler_params=pltpu.CompilerParams(use_tc_tiling_on_sc=False),
        scratch_types=[
            pltpu.VMEM_SHARED((npad, d), jnp.float32),
            pltpu.VMEM_SHARED((npad, ed), jnp.float32),
            pltpu.VMEM_SHARED((npad, LANES), jnp.float32),
            [pltpu.VMEM((CH,), jnp.int32) for _ in range(4)],
            [pltpu.VMEM((CH,), jnp.int32) for _ in range(4)],
            [pltpu.VMEM((CH, d), jnp.float32) for _ in range(2)],
            [pltpu.VMEM((CH, ed), jnp.float32) for _ in range(2)],
            pltpu.VMEM((CH, LANES), jnp.float32),
            [pltpu.SemaphoreType.DMA for _ in range(2)],
            [pltpu.SemaphoreType.DMA for _ in range(2)],
            [pltpu.SemaphoreType.DMA for _ in range(2)],
            pltpu.SemaphoreType.DMA,
        ],
    )
    def sc_fn(x_hbm, i_hbm, j_hbm, attr_hbm, ax_out, s_out, deg_out,
              ax_acc, s_acc, deg_acc, ibuf, jbuf, rows, attrb, onesbuf,
              isem, gsem, osem, sem):
        cid = lax.axis_index("c")
        sid = lax.axis_index("s")
        wid = cid * NS + sid

        zero16 = jnp.zeros((LANES,), jnp.float32)
        ones16 = jnp.ones((LANES,), jnp.float32)

        def zstore(t, carry):
            r = t // (d // LANES)
            c = (t % (d // LANES)) * LANES
            rows[0][r, pl.ds(c, LANES)] = zero16
            return carry

        lax.fori_loop(0, CH * (d // LANES), zstore, 0)

        def zstore_s(r, carry):
            attrb[0][r, pl.ds(0, LANES)] = zero16
            onesbuf[r, pl.ds(0, LANES)] = ones16
            return carry

        lax.fori_loop(0, CH, zstore_s, 0)

        r0 = sid * rpt
        nz = rpt // CH
        for z in range(nz):
            zr0 = r0 + z * CH
            pltpu.async_copy(rows[0], ax_acc.at[pl.ds(zr0, CH), :], sem)
            pltpu.async_copy(attrb[0], s_acc.at[pl.ds(zr0, CH), :], sem)
            pltpu.async_copy(attrb[0], deg_acc.at[pl.ds(zr0, CH), :], sem)
        for z in range(nz):
            zr0 = r0 + z * CH
            pltpu.make_async_copy(
                rows[0], ax_acc.at[pl.ds(zr0, CH), :], sem).wait()
            pltpu.make_async_copy(
                attrb[0], s_acc.at[pl.ds(zr0, CH), :], sem).wait()
            pltpu.make_async_copy(
                attrb[0], deg_acc.at[pl.ds(zr0, CH), :], sem).wait()

        plsc.subcore_barrier()

        ebase = wid * epw

        def issue_idx(k, p4):
            pltpu.async_copy(i_hbm.at[pl.ds(ebase + k * CH, CH)], ibuf[p4],
                             isem[p4 % 2])
            pltpu.async_copy(j_hbm.at[pl.ds(ebase + k * CH, CH)], jbuf[p4],
                             isem[p4 % 2])

        def drain_idx(p4):
            pltpu.make_async_copy(i_hbm.at[pl.ds(0, CH)], ibuf[p4],
                                  isem[p4 % 2]).wait()
            pltpu.make_async_copy(j_hbm.at[pl.ds(0, CH)], jbuf[p4],
                                  isem[p4 % 2]).wait()

        def issue_in(k, p4):
            b = p4 % 2
            pltpu.async_copy(x_hbm.at[jbuf[p4]], rows[b], gsem[b])
            pltpu.async_copy(attr_hbm.at[pl.ds((ebase + k * CH), CH), :],
                             attrb[b], gsem[b])

        def drain_in(p4):
            b = p4 % 2
            pltpu.make_async_copy(x_hbm.at[jbuf[p4]], rows[b],
                                  gsem[b]).wait()
            pltpu.make_async_copy(attr_hbm.at[pl.ds(0, CH), :], attrb[b],
                                  gsem[b]).wait()

        def issue_out(p4):
            b = p4 % 2
            pltpu.async_copy(rows[b], ax_acc.at[ibuf[p4]], osem[b],
                             add=True)
            pltpu.async_copy(attrb[b], s_acc.at[ibuf[p4]], osem[b],
                             add=True)
            pltpu.async_copy(onesbuf, deg_acc.at[ibuf[p4]], osem[b],
                             add=True)

        def drain_out(p4):
            b = p4 % 2
            pltpu.make_async_copy(rows[b], ax_acc.at[ibuf[p4]],
                                  osem[b]).wait()
            pltpu.make_async_copy(attrb[b], s_acc.at[ibuf[p4]],
                                  osem[b]).wait()
            pltpu.make_async_copy(onesbuf, deg_acc.at[ibuf[p4]],
                                  osem[b]).wait()

        issue_idx(0, 0)
        issue_idx(1, 1)
        drain_idx(0)
        issue_in(0, 0)

        def step(it, carry):
            for b4 in range(4):
                k = it * 4 + b4

                @pl.when(k < nch)
                def _():
                    drain_in(b4)
                    issue_out(b4)

                    @pl.when(k + 2 < nch)
                    def _():
                        issue_idx(k + 2, (b4 + 2) % 4)

                    @pl.when(k >= 1)
                    def _():
                        drain_out((b4 - 1) % 4)

                    @pl.when(k + 1 < nch)
                    def _():
                        drain_idx((b4 + 1) % 4)
                        issue_in(k + 1, (b4 + 1) % 4)
            return carry

        lax.fori_loop(0, -(-nch // 4), step, 0)

        drain_out((nch - 1) % 4)

        plsc.subcore_barrier()

        pltpu.async_copy(ax_acc.at[pl.ds(r0, rpt), :],
                         ax_out.at[cid, pl.ds(r0, rpt), :], sem)
        pltpu.async_copy(s_acc.at[pl.ds(r0, rpt), :],
                         s_out.at[cid, pl.ds(r0, rpt), :], sem)
        pltpu.async_copy(deg_acc.at[pl.ds(r0, rpt), :],
                         deg_out.at[cid, pl.ds(r0, rpt), :], sem)
        pltpu.make_async_copy(ax_acc.at[pl.ds(r0, rpt), :],
                              ax_out.at[cid, pl.ds(r0, rpt), :], sem).wait()
        pltpu.make_async_copy(s_acc.at[pl.ds(r0, rpt), :],
                              s_out.at[cid, pl.ds(r0, rpt), :], sem).wait()
        pltpu.make_async_copy(deg_acc.at[pl.ds(r0, rpt), :],
                              deg_out.at[cid, pl.ds(r0, rpt), :], sem).wait()

    return sc_fn(x, ei, ej, attr)


def _tc_dense(eps2, x, axp, sp, degp, we_t, be_r, w1_t, b1r, w2_t, b2r,
              g2, bt2):
    n, d = x.shape
    ed = we_t.shape[0]
    bn = 1000

    def body(eps_ref, x_ref, ax_ref, s_ref, deg_ref, we_ref, be_ref,
             w1_ref, b1_ref, w2_ref, b2_ref, g_ref, bt_ref, o_ref):
        ax = ax_ref[0] + ax_ref[1]
        s = s_ref[0] + s_ref[1]
        deg = (deg_ref[0] + deg_ref[1])[:, 0:1]
        agg = ax + jnp.dot(s, we_ref[...], preferred_element_type=jnp.float32)
        agg = agg + deg * be_ref[...]
        h = x_ref[...] * (1.0 + eps_ref[...]) + agg
        h = jnp.dot(h, w1_ref[...], preferred_element_type=jnp.float32)
        h = jnp.maximum(h + b1_ref[...], 0.0)
        h = jnp.dot(h, w2_ref[...], preferred_element_type=jnp.float32)
        h = h + b2_ref[...]
        mu = jnp.mean(h, axis=-1, keepdims=True)
        hc = h - mu
        var = jnp.mean(hc * hc, axis=-1, keepdims=True)
        o_ref[...] = hc * lax.rsqrt(var + 1e-5) * g_ref[...] + bt_ref[...]

    return pl.pallas_call(
        body,
        grid=(n // bn,),
        in_specs=[
            pl.BlockSpec((1, 1), lambda i: (0, 0)),
            pl.BlockSpec((bn, d), lambda i: (i, 0)),
            pl.BlockSpec((NC, bn, d), lambda i: (0, i, 0)),
            pl.BlockSpec((NC, bn, ed), lambda i: (0, i, 0)),
            pl.BlockSpec((NC, bn, LANES), lambda i: (0, i, 0)),
            pl.BlockSpec((ed, d), lambda i: (0, 0)),
            pl.BlockSpec((1, d), lambda i: (0, 0)),
            pl.BlockSpec((d, d), lambda i: (0, 0)),
            pl.BlockSpec((1, d), lambda i: (0, 0)),
            pl.BlockSpec((d, d), lambda i: (0, 0)),
            pl.BlockSpec((1, d), lambda i: (0, 0)),
            pl.BlockSpec((1, d), lambda i: (0, 0)),
            pl.BlockSpec((1, d), lambda i: (0, 0)),
        ],
        out_specs=pl.BlockSpec((bn, d), lambda i: (i, 0)),
        out_shape=jax.ShapeDtypeStruct((n, d), jnp.float32),
    )(eps2, x, axp, sp, degp, we_t, be_r, w1_t, b1r, w2_t, b2r, g2, bt2)


def kernel(x, edge_index, edge_attr, eps, We, be, W1, b1, W2, b2, gamma, beta):
    n, d = x.shape
    e = edge_index.shape[1]

    ei = edge_index[0].astype(jnp.int32)
    ej = edge_index[1].astype(jnp.int32)
    attr = edge_attr.astype(jnp.float32)

    axp, sp, degp = _sc_aggregate(x, ei, ej, attr, n, d)

    eps2 = jnp.reshape(eps.astype(jnp.float32), (1, 1))
    return _tc_dense(eps2, x, axp, sp, degp, We.T, be[None, :],
                     W1.T, b1[None, :], W2.T, b2[None, :],
                     gamma[None, :], beta[None, :])
